# baseline (speedup 1.0000x reference)
import functools

import jax
import jax.numpy as jnp
from jax import lax
from jax.experimental import pallas as pl
from jax.experimental.pallas import tpu as pltpu
from jax.experimental.pallas import tpu_sc as plsc

_NEG_INF = float("-inf")

T = 2048
D = 768
E = 8
H2 = 1536
TILE = 128
NR = 2 * T + E * TILE
NT = NR // TILE
NTE = 64
TT = 512
NTT = T // TT


def _prep_kernel(x_ref, rw_ref, rb_ref,
                 w_ref, pos_ref, te_ref,
                 ish, tril, carry, base):
    p = pl.program_id(0)
    i = pl.program_id(1)
    tsl = pl.ds(i * TT, TT)

    @pl.when(jnp.logical_and(p == 0, i == 0))
    def _init():
        r = jax.lax.broadcasted_iota(jnp.int32, (TT, TT), 0)
        c = jax.lax.broadcasted_iota(jnp.int32, (TT, TT), 1)
        tril[...] = (r < c).astype(jnp.float32)
        carry[...] = jnp.zeros((E, 1), jnp.float32)

    @pl.when(p == 0)
    def _router():
        lg = jax.lax.dot_general(
            rw_ref[...], x_ref[tsl, :],
            dimension_numbers=(((1,), (1,)), ((), ())),
            preferred_element_type=jnp.float32,
        ) + rb_ref[...][:, None]
        row = jax.lax.broadcasted_iota(jnp.int32, lg.shape, 0)
        m1 = jnp.max(lg, axis=0, keepdims=True)
        i1 = jnp.min(jnp.where(lg == m1, row, E), axis=0, keepdims=True)
        l2 = jnp.where(row == i1, _NEG_INF, lg)
        m2 = jnp.max(l2, axis=0, keepdims=True)
        i2 = jnp.min(jnp.where(l2 == m2, row, E), axis=0, keepdims=True)
        b = jnp.exp(m2 - m1)
        w1 = 1.0 / (1.0 + b)
        ish[0:1, tsl] = i1
        ish[1:2, tsl] = i2
        w_ref[0:1, tsl] = w1
        w_ref[1:2, tsl] = 1.0 - w1
        oh = ((row == i1).astype(jnp.float32)
              + (row == i2).astype(jnp.float32))
        carry[...] += jnp.sum(oh, axis=1, keepdims=True)

    @pl.when(jnp.logical_and(p == 0, i == NTT - 1))
    def _bases():
        ci = carry[...].astype(jnp.int32)
        cp = ((ci + (TILE - 1)) >> 7) << 7
        re = jax.lax.broadcasted_iota(jnp.int32, (E, E), 0)
        ce = jax.lax.broadcasted_iota(jnp.int32, (E, E), 1)
        stril8 = (ce < re).astype(jnp.float32)
        basef = jax.lax.dot_general(
            stril8, cp.astype(jnp.float32),
            dimension_numbers=(((1,), (0,)), ((), ())),
            preferred_element_type=jnp.float32,
        )
        base[...] = basef
        endi = (basef.astype(jnp.int32) + cp) >> 7
        jl = jax.lax.broadcasted_iota(jnp.int32, (1, NTE), 1)
        acc = jnp.zeros((1, NTE), jnp.int32)
        for e in range(E):
            acc = acc + (jl >= endi[e:e + 1, 0:1]).astype(jnp.int32)
        te_ref[...] = jnp.minimum(acc, E - 1)
        carry[...] = basef

    @pl.when(p == 1)
    def _positions():
        i1 = ish[0:1, tsl]
        i2 = ish[1:2, tsl]
        row = jax.lax.broadcasted_iota(jnp.int32, (E, TT), 0)
        oh1 = (row == i1).astype(jnp.float32)
        oh2 = (row == i2).astype(jnp.float32)
        cum1 = jax.lax.dot_general(
            oh1, tril[...],
            dimension_numbers=(((1,), (0,)), ((), ())),
            preferred_element_type=jnp.float32,
        )
        s1 = jnp.sum(oh1, axis=1, keepdims=True)
        cum2 = jax.lax.dot_general(
            oh2, tril[...],
            dimension_numbers=(((1,), (0,)), ((), ())),
            preferred_element_type=jnp.float32,
        ) + s1
        off = carry[...]
        pos1 = jnp.sum(oh1 * (off + cum1), axis=0, keepdims=True)
        pos2 = jnp.sum(oh2 * (off + cum2), axis=0, keepdims=True)
        pos_ref[0:1, tsl] = pos1.astype(jnp.int32)
        pos_ref[1:2, tsl] = pos2.astype(jnp.int32)
        carry[...] = off + s1 + jnp.sum(oh2, axis=1, keepdims=True)


def _prep(x2d, router_w, router_b):
    return pl.pallas_call(
        _prep_kernel,
        grid=(2, NTT),
        in_specs=[
            pl.BlockSpec((T, D), lambda p, i: (0, 0)),
            pl.BlockSpec((E, D), lambda p, i: (0, 0)),
            pl.BlockSpec((E,), lambda p, i: (0,)),
        ],
        out_specs=[
            pl.BlockSpec((2, T), lambda p, i: (0, 0)),
            pl.BlockSpec((2, T), lambda p, i: (0, 0)),
            pl.BlockSpec((1, NTE), lambda p, i: (0, 0)),
        ],
        out_shape=[
            jax.ShapeDtypeStruct((2, T), jnp.float32),
            jax.ShapeDtypeStruct((2, T), jnp.int32),
            jax.ShapeDtypeStruct((1, NTE), jnp.int32),
        ],
        scratch_shapes=[
            pltpu.VMEM((2, T), jnp.int32),
            pltpu.VMEM((TT, TT), jnp.float32),
            pltpu.VMEM((E, 1), jnp.float32),
            pltpu.VMEM((E, 1), jnp.float32),
        ],
        compiler_params=pltpu.CompilerParams(
            dimension_semantics=("arbitrary", "arbitrary"),
        ),
    )(x2d, router_w, router_b)


_PPW = (2 * T) // 16
_GROWS = NR // 16
_GCH = 80


def _dispatch_body(pos_ref, w_ref, x_ref, xg_ref, wrow_ref,
                   posidx, tokidx, wvals, zbuf, zwbuf, idxbuf, rows,
                   srcbuf, srcw, sem):
    cid = lax.axis_index("c")
    sid = lax.axis_index("s")

    @pl.when(cid == 0)
    def _core0():
        lidx = jax.lax.iota(jnp.int32, 16)
        cb = jnp.where(sid < 8, sid, sid - 8) * _PPW

        @pl.when(sid < 8)
        def _():
            pltpu.sync_copy(pos_ref.at[0, pl.ds(sid * _PPW, 128)],
                            posidx.at[0])
            pltpu.sync_copy(pos_ref.at[0, pl.ds(sid * _PPW + 128, 128)],
                            posidx.at[1])
            pltpu.sync_copy(w_ref.at[0, pl.ds(sid * _PPW, 128)],
                            wvals.at[0])
            pltpu.sync_copy(w_ref.at[0, pl.ds(sid * _PPW + 128, 128)],
                            wvals.at[1])

        @pl.when(sid >= 8)
        def _():
            pltpu.sync_copy(pos_ref.at[1, pl.ds((sid - 8) * _PPW, 128)],
                            posidx.at[0])
            pltpu.sync_copy(pos_ref.at[1, pl.ds((sid - 8) * _PPW + 128, 128)],
                            posidx.at[1])
            pltpu.sync_copy(w_ref.at[1, pl.ds((sid - 8) * _PPW, 128)],
                            wvals.at[0])
            pltpu.sync_copy(w_ref.at[1, pl.ds((sid - 8) * _PPW + 128, 128)],
                            wvals.at[1])

        for r in range(2):
            for j in range(8):
                tokidx[r, pl.ds(16 * j, 16)] = cb + 128 * r + 16 * j + lidx

        for j in range(_GROWS // 16):
            zbuf[pl.ds(16 * j, 16)] = jnp.zeros((16,), jnp.int32)
            zwbuf[pl.ds(16 * j, 16)] = jnp.zeros((16,), jnp.float32)
        pltpu.sync_copy(zbuf, srcbuf.at[pl.ds(sid * _GROWS, _GROWS)])
        pltpu.sync_copy(zwbuf, srcw.at[pl.ds(sid * _GROWS, _GROWS)])
        plsc.subcore_barrier()
        for r in range(2):
            pltpu.sync_copy(tokidx.at[r], srcbuf.at[posidx.at[r]])
            pltpu.sync_copy(wvals.at[r], srcw.at[posidx.at[r]])
        plsc.subcore_barrier()

        pltpu.sync_copy(srcw.at[pl.ds(sid * _GROWS, _GROWS)],
                        zwbuf)
        pltpu.sync_copy(zwbuf, wrow_ref.at[pl.ds(sid * _GROWS, _GROWS)])

        for r in range(_GROWS // _GCH):
            base_row = sid * _GROWS + r * _GCH
            pltpu.sync_copy(srcbuf.at[pl.ds(base_row, _GCH)], idxbuf.at[r])
            pltpu.async_copy(x_ref.at[idxbuf.at[r]], rows, sem).wait()
            pltpu.sync_copy(rows, xg_ref.at[pl.ds(base_row, _GCH)])


def _dispatch(pos2t, wcat, x2d):
    mesh = plsc.VectorSubcoreMesh(core_axis_name="c", subcore_axis_name="s")
    fn = functools.partial(
        pl.kernel,
        mesh=mesh,
        out_type=[
            jax.ShapeDtypeStruct((NR, D), jnp.float32),
            jax.ShapeDtypeStruct((NR,), jnp.float32),
        ],
        scratch_types=[
            pltpu.VMEM((2, 128), jnp.int32),
            pltpu.VMEM((2, 128), jnp.int32),
            pltpu.VMEM((2, 128), jnp.float32),
            pltpu.VMEM((_GROWS,), jnp.int32),
            pltpu.VMEM((_GROWS,), jnp.float32),
            pltpu.VMEM((_GROWS // _GCH, _GCH), jnp.int32),
            pltpu.VMEM((_GCH, D), jnp.float32),
            pltpu.VMEM_SHARED((NR,), jnp.int32),
            pltpu.VMEM_SHARED((NR,), jnp.float32),
            pltpu.SemaphoreType.DMA,
        ],
    )(_dispatch_body)
    return fn(pos2t, wcat, x2d)


def _ffn_kernel(te_ref, xg_ref, wrow_ref, fc1w_ref, fc1b_ref,
                fc2w_ref, fc2b_ref, yg_ref, w1b_scr, w2b_scr):
    i = pl.program_id(0)
    e = te_ref[i]
    prev = te_ref[jnp.maximum(i - 1, 0)]

    @pl.when(jnp.logical_or(i == 0, e != prev))
    def _cache():
        w1b_scr[...] = fc1w_ref[0].astype(jnp.bfloat16)
        w2b_scr[...] = fc2w_ref[0].astype(jnp.bfloat16)

    h = jax.lax.dot_general(
        xg_ref[...].astype(jnp.bfloat16), w1b_scr[...],
        dimension_numbers=(((1,), (1,)), ((), ())),
        preferred_element_type=jnp.float32,
    ) + fc1b_ref[pl.ds(e, 1), :]
    h1 = h[:, :H2 // 2]
    h2 = h[:, H2 // 2:]
    g = h1 * jax.nn.sigmoid(h1) * h2
    y = jax.lax.dot_general(
        g.astype(jnp.bfloat16), w2b_scr[...],
        dimension_numbers=(((1,), (1,)), ((), ())),
        preferred_element_type=jnp.float32,
    ) + fc2b_ref[pl.ds(e, 1), :]
    yg_ref[...] = y * wrow_ref[...]


def _ffn(te, xg, wrow, fc1_w, fc1_b, fc2_w, fc2_b):
    grid_spec = pltpu.PrefetchScalarGridSpec(
        num_scalar_prefetch=1,
        grid=(NT,),
        in_specs=[
            pl.BlockSpec((TILE, D), lambda i, te: (i, 0)),
            pl.BlockSpec((TILE, 1), lambda i, te: (i, 0)),
            pl.BlockSpec((1, H2, D), lambda i, te: (te[i], 0, 0)),
            pl.BlockSpec((E, H2), lambda i, te: (0, 0)),
            pl.BlockSpec((1, D, H2 // 2), lambda i, te: (te[i], 0, 0)),
            pl.BlockSpec((E, D), lambda i, te: (0, 0)),
        ],
        out_specs=pl.BlockSpec((TILE, D), lambda i, te: (i, 0)),
        scratch_shapes=[
            pltpu.VMEM((H2, D), jnp.bfloat16),
            pltpu.VMEM((D, H2 // 2), jnp.bfloat16),
        ],
    )
    return pl.pallas_call(
        _ffn_kernel,
        grid_spec=grid_spec,
        out_shape=jax.ShapeDtypeStruct((NR, D), jnp.float32),
        compiler_params=pltpu.CompilerParams(
            dimension_semantics=("arbitrary",),
        ),
    )(te, xg, wrow, fc1_w, fc1_b, fc2_w, fc2_b)


_TPW = T // 32


def _combine_body(yg_ref, pos_ref, out_ref, pidx, rowsA, rowsB, sem):
    cid = lax.axis_index("c")
    sid = lax.axis_index("s")
    wid = sid * 2 + cid
    t0 = wid * _TPW

    pltpu.sync_copy(pos_ref.at[0, pl.ds(t0, _TPW)], pidx.at[0])
    pltpu.sync_copy(pos_ref.at[1, pl.ds(t0, _TPW)], pidx.at[1])
    pltpu.async_copy(yg_ref.at[pidx.at[0]], rowsA, sem).wait()
    pltpu.async_copy(yg_ref.at[pidx.at[1]], rowsB, sem).wait()

    def body(t, carry):
        for k in range(D // 16):
            a = rowsA[t, pl.ds(16 * k, 16)]
            b = rowsB[t, pl.ds(16 * k, 16)]
            rowsA[t, pl.ds(16 * k, 16)] = a + b
        return carry

    jax.lax.fori_loop(0, _TPW, body, 0)
    pltpu.sync_copy(rowsA, out_ref.at[pl.ds(t0, _TPW)])


def _combine(yg, pos2t):
    mesh = plsc.VectorSubcoreMesh(core_axis_name="c", subcore_axis_name="s")
    fn = functools.partial(
        pl.kernel,
        mesh=mesh,
        out_type=jax.ShapeDtypeStruct((T, D), jnp.float32),
        scratch_types=[
            pltpu.VMEM((2, _TPW), jnp.int32),
            pltpu.VMEM((_TPW, D), jnp.float32),
            pltpu.VMEM((_TPW, D), jnp.float32),
            pltpu.SemaphoreType.DMA,
        ],
    )(_combine_body)
    return fn(yg, pos2t)


def kernel(x, router_w, router_b, fc1_w, fc1_b, fc2_w, fc2_b):
    B = x.shape[0]
    x2d = x.reshape(T, D)
    wcat, pos, te = _prep(x2d, router_w, router_b)
    xg, wrow = _dispatch(pos, wcat, x2d)
    yg = _ffn(te.reshape(NTE), xg, wrow.reshape(NR, 1),
              fc1_w, fc1_b, fc2_w, fc2_b)
    out = _combine(yg, pos)
    return out.reshape(B, T, D)


# speedup vs baseline: 1.2798x; 1.2798x over previous
import functools

import jax
import jax.numpy as jnp
from jax import lax
from jax.experimental import pallas as pl
from jax.experimental.pallas import tpu as pltpu
from jax.experimental.pallas import tpu_sc as plsc

_NEG_INF = float("-inf")

T = 2048
D = 768
E = 8
H2 = 1536
TILE = 128
NR = 2 * T + E * TILE
NT = NR // TILE
NTE = 64
TT = 512
NTT = T // TT


def _prep_kernel(x_ref, rw_ref, rb_ref,
                 w_ref, pos_ref, te_ref,
                 ish, tril, carry, base):
    p = pl.program_id(0)
    i = pl.program_id(1)
    tsl = pl.ds(i * TT, TT)

    @pl.when(jnp.logical_and(p == 0, i == 0))
    def _init():
        r = jax.lax.broadcasted_iota(jnp.int32, (TT, TT), 0)
        c = jax.lax.broadcasted_iota(jnp.int32, (TT, TT), 1)
        tril[...] = (r < c).astype(jnp.float32)
        carry[...] = jnp.zeros((E, 1), jnp.float32)

    @pl.when(p == 0)
    def _router():
        lg = jax.lax.dot_general(
            rw_ref[...], x_ref[tsl, :],
            dimension_numbers=(((1,), (1,)), ((), ())),
            preferred_element_type=jnp.float32,
        ) + rb_ref[...][:, None]
        row = jax.lax.broadcasted_iota(jnp.int32, lg.shape, 0)
        m1 = jnp.max(lg, axis=0, keepdims=True)
        i1 = jnp.min(jnp.where(lg == m1, row, E), axis=0, keepdims=True)
        l2 = jnp.where(row == i1, _NEG_INF, lg)
        m2 = jnp.max(l2, axis=0, keepdims=True)
        i2 = jnp.min(jnp.where(l2 == m2, row, E), axis=0, keepdims=True)
        b = jnp.exp(m2 - m1)
        w1 = 1.0 / (1.0 + b)
        ish[0:1, tsl] = i1
        ish[1:2, tsl] = i2
        w_ref[0:1, tsl] = w1
        w_ref[1:2, tsl] = 1.0 - w1
        oh = ((row == i1).astype(jnp.float32)
              + (row == i2).astype(jnp.float32))
        carry[...] += jnp.sum(oh, axis=1, keepdims=True)

    @pl.when(jnp.logical_and(p == 0, i == NTT - 1))
    def _bases():
        ci = carry[...].astype(jnp.int32)
        cp = ((ci + (TILE - 1)) >> 7) << 7
        re = jax.lax.broadcasted_iota(jnp.int32, (E, E), 0)
        ce = jax.lax.broadcasted_iota(jnp.int32, (E, E), 1)
        stril8 = (ce < re).astype(jnp.float32)
        basef = jax.lax.dot_general(
            stril8, cp.astype(jnp.float32),
            dimension_numbers=(((1,), (0,)), ((), ())),
            preferred_element_type=jnp.float32,
        )
        base[...] = basef
        endi = (basef.astype(jnp.int32) + cp) >> 7
        jl = jax.lax.broadcasted_iota(jnp.int32, (1, NTE), 1)
        acc = jnp.zeros((1, NTE), jnp.int32)
        for e in range(E):
            acc = acc + (jl >= endi[e:e + 1, 0:1]).astype(jnp.int32)
        te_ref[...] = jnp.minimum(acc, E - 1)
        carry[...] = basef

    @pl.when(p == 1)
    def _positions():
        i1 = ish[0:1, tsl]
        i2 = ish[1:2, tsl]
        row = jax.lax.broadcasted_iota(jnp.int32, (E, TT), 0)
        oh1 = (row == i1).astype(jnp.float32)
        oh2 = (row == i2).astype(jnp.float32)
        cum1 = jax.lax.dot_general(
            oh1, tril[...],
            dimension_numbers=(((1,), (0,)), ((), ())),
            preferred_element_type=jnp.float32,
        )
        s1 = jnp.sum(oh1, axis=1, keepdims=True)
        cum2 = jax.lax.dot_general(
            oh2, tril[...],
            dimension_numbers=(((1,), (0,)), ((), ())),
            preferred_element_type=jnp.float32,
        ) + s1
        off = carry[...]
        pos1 = jnp.sum(oh1 * (off + cum1), axis=0, keepdims=True)
        pos2 = jnp.sum(oh2 * (off + cum2), axis=0, keepdims=True)
        pos_ref[0:1, tsl] = pos1.astype(jnp.int32)
        pos_ref[1:2, tsl] = pos2.astype(jnp.int32)
        carry[...] = off + s1 + jnp.sum(oh2, axis=1, keepdims=True)


def _prep(x2d, router_w, router_b):
    return pl.pallas_call(
        _prep_kernel,
        grid=(2, NTT),
        in_specs=[
            pl.BlockSpec((T, D), lambda p, i: (0, 0)),
            pl.BlockSpec((E, D), lambda p, i: (0, 0)),
            pl.BlockSpec((E,), lambda p, i: (0,)),
        ],
        out_specs=[
            pl.BlockSpec((2, T), lambda p, i: (0, 0)),
            pl.BlockSpec((2, T), lambda p, i: (0, 0)),
            pl.BlockSpec((1, NTE), lambda p, i: (0, 0)),
        ],
        out_shape=[
            jax.ShapeDtypeStruct((2, T), jnp.float32),
            jax.ShapeDtypeStruct((2, T), jnp.int32),
            jax.ShapeDtypeStruct((1, NTE), jnp.int32),
        ],
        scratch_shapes=[
            pltpu.VMEM((2, T), jnp.int32),
            pltpu.VMEM((TT, TT), jnp.float32),
            pltpu.VMEM((E, 1), jnp.float32),
            pltpu.VMEM((E, 1), jnp.float32),
        ],
        compiler_params=pltpu.CompilerParams(
            dimension_semantics=("arbitrary", "arbitrary"),
        ),
    )(x2d, router_w, router_b)


_TPW = T // 32


def _dispatch_body(pos_ref, w_ref, x_ref, xg_ref, wrow_ref,
                   pidx, wvals, xrows, sem):
    cid = lax.axis_index("c")
    sid = lax.axis_index("s")
    t0 = (sid * 2 + cid) * _TPW

    pltpu.sync_copy(pos_ref.at[0, pl.ds(t0, _TPW)], pidx.at[0])
    pltpu.sync_copy(pos_ref.at[1, pl.ds(t0, _TPW)], pidx.at[1])
    pltpu.sync_copy(w_ref.at[0, pl.ds(t0, _TPW)], wvals.at[0])
    pltpu.sync_copy(w_ref.at[1, pl.ds(t0, _TPW)], wvals.at[1])
    pltpu.sync_copy(x_ref.at[pl.ds(t0, _TPW)], xrows)
    c1 = pltpu.async_copy(xrows, xg_ref.at[pidx.at[0]], sem)
    c2 = pltpu.async_copy(xrows, xg_ref.at[pidx.at[1]], sem)
    c3 = pltpu.async_copy(wvals.at[0], wrow_ref.at[pidx.at[0]], sem)
    c4 = pltpu.async_copy(wvals.at[1], wrow_ref.at[pidx.at[1]], sem)
    c1.wait()
    c2.wait()
    c3.wait()
    c4.wait()


def _dispatch(pos2t, wcat, x2d):
    mesh = plsc.VectorSubcoreMesh(core_axis_name="c", subcore_axis_name="s")
    fn = functools.partial(
        pl.kernel,
        mesh=mesh,
        out_type=[
            jax.ShapeDtypeStruct((NR, D), jnp.float32),
            jax.ShapeDtypeStruct((NR,), jnp.float32),
        ],
        scratch_types=[
            pltpu.VMEM((2, _TPW), jnp.int32),
            pltpu.VMEM((2, _TPW), jnp.float32),
            pltpu.VMEM((_TPW, D), jnp.float32),
            pltpu.SemaphoreType.DMA,
        ],
    )(_dispatch_body)
    return fn(pos2t, wcat, x2d)


def _ffn_kernel(te_ref, xg_ref, wrow_ref, fc1w_ref, fc1b_ref,
                fc2w_ref, fc2b_ref, yg_ref, w1b_scr, w2b_scr):
    i = pl.program_id(0)
    e = te_ref[i]
    prev = te_ref[jnp.maximum(i - 1, 0)]

    @pl.when(jnp.logical_or(i == 0, e != prev))
    def _cache():
        w1b_scr[...] = fc1w_ref[0].astype(jnp.bfloat16)
        w2b_scr[...] = fc2w_ref[0].astype(jnp.bfloat16)

    h = jax.lax.dot_general(
        xg_ref[...].astype(jnp.bfloat16), w1b_scr[...],
        dimension_numbers=(((1,), (1,)), ((), ())),
        preferred_element_type=jnp.float32,
    ) + fc1b_ref[pl.ds(e, 1), :]
    h1 = h[:, :H2 // 2]
    h2 = h[:, H2 // 2:]
    g = h1 * jax.nn.sigmoid(h1) * h2
    y = jax.lax.dot_general(
        g.astype(jnp.bfloat16), w2b_scr[...],
        dimension_numbers=(((1,), (1,)), ((), ())),
        preferred_element_type=jnp.float32,
    ) + fc2b_ref[pl.ds(e, 1), :]
    yg_ref[...] = y * wrow_ref[...]


def _ffn(te, xg, wrow, fc1_w, fc1_b, fc2_w, fc2_b):
    grid_spec = pltpu.PrefetchScalarGridSpec(
        num_scalar_prefetch=1,
        grid=(NT,),
        in_specs=[
            pl.BlockSpec((TILE, D), lambda i, te: (i, 0)),
            pl.BlockSpec((TILE, 1), lambda i, te: (i, 0)),
            pl.BlockSpec((1, H2, D), lambda i, te: (te[i], 0, 0)),
            pl.BlockSpec((E, H2), lambda i, te: (0, 0)),
            pl.BlockSpec((1, D, H2 // 2), lambda i, te: (te[i], 0, 0)),
            pl.BlockSpec((E, D), lambda i, te: (0, 0)),
        ],
        out_specs=pl.BlockSpec((TILE, D), lambda i, te: (i, 0)),
        scratch_shapes=[
            pltpu.VMEM((H2, D), jnp.bfloat16),
            pltpu.VMEM((D, H2 // 2), jnp.bfloat16),
        ],
    )
    return pl.pallas_call(
        _ffn_kernel,
        grid_spec=grid_spec,
        out_shape=jax.ShapeDtypeStruct((NR, D), jnp.float32),
        compiler_params=pltpu.CompilerParams(
            dimension_semantics=("arbitrary",),
        ),
    )(te, xg, wrow, fc1_w, fc1_b, fc2_w, fc2_b)


def _combine_body(yg_ref, pos_ref, out_ref, pidx, rowsA, rowsB, sem):
    cid = lax.axis_index("c")
    sid = lax.axis_index("s")
    wid = sid * 2 + cid
    t0 = wid * _TPW

    pltpu.sync_copy(pos_ref.at[0, pl.ds(t0, _TPW)], pidx.at[0])
    pltpu.sync_copy(pos_ref.at[1, pl.ds(t0, _TPW)], pidx.at[1])
    cA = pltpu.async_copy(yg_ref.at[pidx.at[0]], rowsA, sem)
    cB = pltpu.async_copy(yg_ref.at[pidx.at[1]], rowsB, sem)
    cA.wait()
    cB.wait()

    def body(t, carry):
        for k in range(D // 16):
            a = rowsA[t, pl.ds(16 * k, 16)]
            b = rowsB[t, pl.ds(16 * k, 16)]
            rowsA[t, pl.ds(16 * k, 16)] = a + b
        return carry

    jax.lax.fori_loop(0, _TPW, body, 0)
    pltpu.sync_copy(rowsA, out_ref.at[pl.ds(t0, _TPW)])


def _combine(yg, pos2t):
    mesh = plsc.VectorSubcoreMesh(core_axis_name="c", subcore_axis_name="s")
    fn = functools.partial(
        pl.kernel,
        mesh=mesh,
        out_type=jax.ShapeDtypeStruct((T, D), jnp.float32),
        scratch_types=[
            pltpu.VMEM((2, _TPW), jnp.int32),
            pltpu.VMEM((_TPW, D), jnp.float32),
            pltpu.VMEM((_TPW, D), jnp.float32),
            pltpu.SemaphoreType.DMA,
        ],
    )(_combine_body)
    return fn(yg, pos2t)


def kernel(x, router_w, router_b, fc1_w, fc1_b, fc2_w, fc2_b):
    B = x.shape[0]
    x2d = x.reshape(T, D)
    wcat, pos, te = _prep(x2d, router_w, router_b)
    xg, wrow = _dispatch(pos, wcat, x2d)
    yg = _ffn(te.reshape(NTE), xg, wrow.reshape(NR, 1),
              fc1_w, fc1_b, fc2_w, fc2_b)
    out = _combine(yg, pos)
    return out.reshape(B, T, D)


# speedup vs baseline: 2.4013x; 1.8763x over previous
import functools

import jax
import jax.numpy as jnp
from jax.experimental import pallas as pl
from jax.experimental.pallas import tpu as pltpu

_NEG_INF = float("-inf")


def _moe_dense_kernel(x_ref, rw_ref, rb_ref, fc1w_ref, fc1b_ref,
                      fc2w_ref, fc2b_ref, out_ref, cw_ref, *, n_experts):
    e = pl.program_id(0)

    @pl.when(e == 0)
    def _router():
        logits = jax.lax.dot_general(
            x_ref[0], rw_ref[...],
            dimension_numbers=(((1,), (1,)), ((), ())),
            preferred_element_type=jnp.float32,
        ) + rb_ref[...][None, :]
        col = jax.lax.broadcasted_iota(jnp.int32, logits.shape, 1)
        m1 = jnp.max(logits, axis=1, keepdims=True)
        i1 = jnp.min(jnp.where(logits == m1, col, n_experts), axis=1,
                     keepdims=True)
        l2 = jnp.where(col == i1, _NEG_INF, logits)
        m2 = jnp.max(l2, axis=1, keepdims=True)
        i2 = jnp.min(jnp.where(l2 == m2, col, n_experts), axis=1,
                     keepdims=True)
        b = jnp.exp(m2 - m1)
        w1 = 1.0 / (1.0 + b)
        w2 = 1.0 - w1
        cw_ref[...] = (w1 * (col == i1).astype(jnp.float32)
                       + w2 * (col == i2).astype(jnp.float32))

    col_t = jax.lax.broadcasted_iota(jnp.int32, cw_ref.shape, 1)
    cw = jnp.sum(jnp.where(col_t == e, cw_ref[...], 0.0), axis=1,
                 keepdims=True)

    h = (jax.lax.dot_general(
        x_ref[0].astype(jnp.bfloat16), fc1w_ref[0].astype(jnp.bfloat16),
        dimension_numbers=(((1,), (1,)), ((), ())),
        preferred_element_type=jnp.float32,
    ) + fc1b_ref[pl.ds(e, 1), :]).astype(jnp.bfloat16)
    hdim = h.shape[1] // 2
    h1 = h[:, :hdim]
    h2 = h[:, hdim:]
    g = h1 * jax.nn.sigmoid(h1) * h2
    y = jax.lax.dot_general(
        cw.astype(jnp.bfloat16) * g, fc2w_ref[0].astype(jnp.bfloat16),
        dimension_numbers=(((1,), (1,)), ((), ())),
        preferred_element_type=jnp.float32,
    ) + cw * fc2b_ref[pl.ds(e, 1), :]

    @pl.when(e == 0)
    def _init():
        out_ref[0] = y

    @pl.when(e != 0)
    def _acc():
        out_ref[0] += y


def kernel(x, router_w, router_b, fc1_w, fc1_b, fc2_w, fc2_b):
    B, T, D = x.shape
    E, H2, _ = fc1_w.shape

    return pl.pallas_call(
        functools.partial(_moe_dense_kernel, n_experts=E),
        grid=(E,),
        in_specs=[
            pl.BlockSpec((B, T, D), lambda e: (0, 0, 0)),
            pl.BlockSpec((E, D), lambda e: (0, 0)),
            pl.BlockSpec((E,), lambda e: (0,)),
            pl.BlockSpec((1, H2, D), lambda e: (e, 0, 0)),
            pl.BlockSpec((E, H2), lambda e: (0, 0)),
            pl.BlockSpec((1, D, H2 // 2), lambda e: (e, 0, 0)),
            pl.BlockSpec((E, D), lambda e: (0, 0)),
        ],
        out_specs=pl.BlockSpec((B, T, D), lambda e: (0, 0, 0)),
        out_shape=jax.ShapeDtypeStruct((B, T, D), x.dtype),
        scratch_shapes=[
            pltpu.VMEM((T, E), jnp.float32),
        ],
        compiler_params=pltpu.CompilerParams(
            dimension_semantics=("arbitrary",),
        ),
    )(x, router_w, router_b, fc1_w, fc1_b, fc2_w, fc2_b)
